# baseline (speedup 1.0000x reference)
import functools

import jax
import jax.numpy as jnp
from jax import lax
from jax.experimental import pallas as pl
from jax.experimental.pallas import tpu as pltpu
from jax.experimental.pallas import tpu_sc as plsc

NUM_VTX = 227
NUM_DIM = 3
EMB = 128
BATCH = 256
SEQ = 2048
SCALE = float(EMB) ** 0.5

NC = 2
NS = 16
NW = NC * NS
S_BLK = 128
B_BLK = BATCH // 2
NBUF = 4


def _body(vert_hbm, vtx_hbm, pos_hbm, dim_hbm, out_hbm,
          idx1, table1, dim_v, comb_v, out_v, isem, o0, o1, o2, o3):
    osem = [o0, o1, o2, o3]
    wid = lax.axis_index("s") * NC + lax.axis_index("c")
    j = lax.rem(wid, 16)
    h = wid // 16
    s0 = j * S_BLK
    b0 = h * B_BLK

    pltpu.sync_copy(vtx_hbm, table1)
    pltpu.sync_copy(dim_hbm, dim_v)
    pltpu.sync_copy(pos_hbm.at[pl.ds(s0, S_BLK)], comb_v)

    def idx_row(b, carry):
        pltpu.async_copy(
            vert_hbm.at[pl.ds((b0 + b) * SEQ + s0, S_BLK)],
            idx1.at[pl.ds(b * S_BLK, S_BLK)], isem)
        return carry
    lax.fori_loop(0, B_BLK, idx_row, 0)

    def idx_drain(b, carry):
        pltpu.make_async_copy(
            vert_hbm.at[pl.ds(b0 * SEQ + s0, S_BLK)],
            idx1.at[pl.ds(0, S_BLK)], isem).wait()
        return carry
    lax.fori_loop(0, B_BLK, idx_drain, 0)

    def comb_row(i, carry):
        r = lax.rem(s0 + i, NUM_DIM)
        w0 = jnp.full((16,), jnp.where(r == 0, SCALE, 0.0), jnp.float32)
        w1 = jnp.full((16,), jnp.where(r == 1, SCALE, 0.0), jnp.float32)
        w2 = jnp.full((16,), jnp.where(r == 2, SCALE, 0.0), jnp.float32)
        for g in range(EMB // 16):
            sl = pl.ds(g * 16, 16)
            comb_v[i, sl] = (comb_v[i, sl] * SCALE
                             + dim_v[0, sl] * w0
                             + dim_v[1, sl] * w1
                             + dim_v[2, sl] * w2)
        return carry
    lax.fori_loop(0, S_BLK, comb_row, 0)

    col = lax.iota(jnp.int32, 16)

    def start_out(b, k):
        pltpu.async_copy(
            out_v.at[k], out_hbm.at[b0 + b, pl.ds(s0, S_BLK)], osem[k])

    def wait_out(b, k):
        pltpu.make_async_copy(
            out_v.at[k],
            out_hbm.at[b0 + b, pl.ds(s0, S_BLK)], osem[k]).wait()

    def batch_pair(bp, carry):
        for k in range(NBUF):
            b = bp * NBUF + k

            @pl.when(b >= NBUF)
            def _():
                wait_out(b - NBUF, k)

            @plsc.parallel_loop(0, S_BLK, step=1, unroll=16)
            def seq_row(i):
                vvec = plsc.load_gather(
                    idx1, [jnp.full((16,), b * S_BLK + i, jnp.int32)])
                base = vvec * EMB + col
                for g in range(EMB // 16):
                    sl = pl.ds(g * 16, 16)
                    row = plsc.load_gather(table1, [base + g * 16])
                    out_v[k, i, sl] = row + comb_v[i, sl]
            start_out(b, k)
        return carry
    lax.fori_loop(0, B_BLK // NBUF, batch_pair, 0)

    for k in range(NBUF):
        wait_out(B_BLK - NBUF + k, k)


@jax.jit
def kernel(vertices, vtx_table, pos_table, dim_table):
    vert1 = vertices.astype(jnp.int32).reshape(-1)
    vtx1 = (vtx_table * SCALE).reshape(-1)
    mesh = plsc.VectorSubcoreMesh(core_axis_name="c", subcore_axis_name="s")
    f = functools.partial(
        pl.kernel,
        mesh=mesh,
        out_type=jax.ShapeDtypeStruct((BATCH, SEQ, EMB), jnp.float32),
        scratch_types=[
            pltpu.VMEM((B_BLK * S_BLK,), jnp.int32),
            pltpu.VMEM((NUM_VTX * EMB,), jnp.float32),
            pltpu.VMEM((NUM_DIM, EMB), jnp.float32),
            pltpu.VMEM((S_BLK, EMB), jnp.float32),
            pltpu.VMEM((NBUF, S_BLK, EMB), jnp.float32),
        ] + [pltpu.SemaphoreType.DMA] * 5,
        compiler_params=pltpu.CompilerParams(needs_layout_passes=False),
    )(_body)
    return f(vert1, vtx1, pos_table, dim_table)


# speedup vs baseline: 1.0334x; 1.0334x over previous
import functools

import jax
import jax.numpy as jnp
from jax import lax
from jax.experimental import pallas as pl
from jax.experimental.pallas import tpu as pltpu
from jax.experimental.pallas import tpu_sc as plsc

NUM_VTX = 227
NUM_DIM = 3
EMB = 128
BATCH = 256
SEQ = 2048
SCALE = float(EMB) ** 0.5

NC = 2
NS = 16
NW = NC * NS
S_BLK = 128
B_BLK = BATCH // 2
NBUF = 2


def _body(vert_hbm, vtx_hbm, pos_hbm, dim_hbm, out_hbm,
          idx1, table1, dim_v, comb_v, out_v, isem, o0, o1):
    osem = [o0, o1]
    wid = lax.axis_index("s") * NC + lax.axis_index("c")
    j = lax.rem(wid, 16)
    h = wid // 16
    s0 = j * S_BLK
    b0 = h * B_BLK

    pltpu.sync_copy(vtx_hbm, table1)
    pltpu.sync_copy(dim_hbm, dim_v)
    pltpu.sync_copy(pos_hbm.at[pl.ds(s0, S_BLK)], comb_v)

    def idx_row(b, carry):
        pltpu.async_copy(
            vert_hbm.at[pl.ds((b0 + b) * SEQ + s0, S_BLK)],
            idx1.at[pl.ds(b * S_BLK, S_BLK)], isem)
        return carry
    lax.fori_loop(0, B_BLK, idx_row, 0)

    def idx_drain(b, carry):
        pltpu.make_async_copy(
            vert_hbm.at[pl.ds(b0 * SEQ + s0, S_BLK)],
            idx1.at[pl.ds(0, S_BLK)], isem).wait()
        return carry
    lax.fori_loop(0, B_BLK, idx_drain, 0)

    def comb_row(i, carry):
        r = lax.rem(s0 + i, NUM_DIM)
        w0 = jnp.full((16,), jnp.where(r == 0, SCALE, 0.0), jnp.float32)
        w1 = jnp.full((16,), jnp.where(r == 1, SCALE, 0.0), jnp.float32)
        w2 = jnp.full((16,), jnp.where(r == 2, SCALE, 0.0), jnp.float32)
        for g in range(EMB // 16):
            sl = pl.ds(g * 16, 16)
            comb_v[i, sl] = (comb_v[i, sl] * SCALE
                             + dim_v[0, sl] * w0
                             + dim_v[1, sl] * w1
                             + dim_v[2, sl] * w2)
        return carry
    lax.fori_loop(0, S_BLK, comb_row, 0)

    col = lax.iota(jnp.int32, 16)

    def start_out(b, k):
        pltpu.async_copy(
            out_v.at[k], out_hbm.at[b0 + b, pl.ds(s0, S_BLK)], osem[k])

    def wait_out(b, k):
        pltpu.make_async_copy(
            out_v.at[k],
            out_hbm.at[b0 + b, pl.ds(s0, S_BLK)], osem[k]).wait()

    def batch_pair(bp, carry):
        for k in range(NBUF):
            b = bp * NBUF + k

            @pl.when(b >= NBUF)
            def _():
                wait_out(b - NBUF, k)

            @plsc.parallel_loop(0, S_BLK, step=1, unroll=8)
            def seq_row(i):
                vvec = plsc.load_gather(
                    idx1, [jnp.full((16,), b * S_BLK + i, jnp.int32)])
                base = vvec * EMB + col
                for g in range(EMB // 16):
                    sl = pl.ds(g * 16, 16)
                    row = plsc.load_gather(table1, [base + g * 16])
                    out_v[k, i, sl] = row + comb_v[i, sl]
            start_out(b, k)
        return carry
    lax.fori_loop(0, B_BLK // NBUF, batch_pair, 0)

    for k in range(NBUF):
        wait_out(B_BLK - NBUF + k, k)


@jax.jit
def kernel(vertices, vtx_table, pos_table, dim_table):
    vert1 = vertices.astype(jnp.int32).reshape(-1)
    vtx1 = (vtx_table * SCALE).reshape(-1)
    mesh = plsc.VectorSubcoreMesh(core_axis_name="c", subcore_axis_name="s")
    f = functools.partial(
        pl.kernel,
        mesh=mesh,
        out_type=jax.ShapeDtypeStruct((BATCH, SEQ, EMB), jnp.float32),
        scratch_types=[
            pltpu.VMEM((B_BLK * S_BLK,), jnp.int32),
            pltpu.VMEM((NUM_VTX * EMB,), jnp.float32),
            pltpu.VMEM((NUM_DIM, EMB), jnp.float32),
            pltpu.VMEM((S_BLK, EMB), jnp.float32),
            pltpu.VMEM((NBUF, S_BLK, EMB), jnp.float32),
        ] + [pltpu.SemaphoreType.DMA] * 3,
        compiler_params=pltpu.CompilerParams(needs_layout_passes=False),
    )(_body)
    return f(vert1, vtx1, pos_table, dim_table)


# speedup vs baseline: 1.0352x; 1.0018x over previous
import functools

import jax
import jax.numpy as jnp
from jax import lax
from jax.experimental import pallas as pl
from jax.experimental.pallas import tpu as pltpu
from jax.experimental.pallas import tpu_sc as plsc

NUM_VTX = 227
NUM_DIM = 3
EMB = 128
BATCH = 256
SEQ = 2048
SCALE = float(EMB) ** 0.5

NC = 2
NS = 16
NW = NC * NS
S_BLK = 128
B_BLK = BATCH // 2
NBUF = 2
BPB = 2
NPAIR = B_BLK // BPB


def _body(vert_hbm, vtx_hbm, pos_hbm, dim_hbm, out_hbm,
          idx1, table1, dim_v, comb_v, out_v, isem, o0, o1):
    osem = [o0, o1]
    wid = lax.axis_index("s") * NC + lax.axis_index("c")
    j = lax.rem(wid, 16)
    h = wid // 16
    s0 = j * S_BLK
    b0 = h * B_BLK

    pltpu.sync_copy(vtx_hbm, table1)
    pltpu.sync_copy(dim_hbm, dim_v)
    pltpu.sync_copy(pos_hbm.at[pl.ds(s0, S_BLK)], comb_v)

    def idx_row(b, carry):
        pltpu.async_copy(
            vert_hbm.at[pl.ds((b0 + b) * SEQ + s0, S_BLK)],
            idx1.at[pl.ds(b * S_BLK, S_BLK)], isem)
        return carry
    lax.fori_loop(0, B_BLK, idx_row, 0)

    def idx_drain(b, carry):
        pltpu.make_async_copy(
            vert_hbm.at[pl.ds(b0 * SEQ + s0, S_BLK)],
            idx1.at[pl.ds(0, S_BLK)], isem).wait()
        return carry
    lax.fori_loop(0, B_BLK, idx_drain, 0)

    def comb_row(i, carry):
        r = lax.rem(s0 + i, NUM_DIM)
        w0 = jnp.full((16,), jnp.where(r == 0, SCALE, 0.0), jnp.float32)
        w1 = jnp.full((16,), jnp.where(r == 1, SCALE, 0.0), jnp.float32)
        w2 = jnp.full((16,), jnp.where(r == 2, SCALE, 0.0), jnp.float32)
        for g in range(EMB // 16):
            sl = pl.ds(g * 16, 16)
            comb_v[i, sl] = (comb_v[i, sl] * SCALE
                             + dim_v[0, sl] * w0
                             + dim_v[1, sl] * w1
                             + dim_v[2, sl] * w2)
        return carry
    lax.fori_loop(0, S_BLK, comb_row, 0)

    col = lax.iota(jnp.int32, 16)

    def start_out(p, k):
        for r in range(BPB):
            pltpu.async_copy(
                out_v.at[k, pl.ds(r * S_BLK, S_BLK)],
                out_hbm.at[b0 + p * BPB + r, pl.ds(s0, S_BLK)], osem[k])

    def wait_out(p, k):
        for r in range(BPB):
            pltpu.make_async_copy(
                out_v.at[k, pl.ds(r * S_BLK, S_BLK)],
                out_hbm.at[b0 + p * BPB + r, pl.ds(s0, S_BLK)],
                osem[k]).wait()

    def batch_pair(bp, carry):
        for k in range(NBUF):
            p = bp * NBUF + k

            @pl.when(p >= NBUF)
            def _():
                wait_out(p - NBUF, k)

            @plsc.parallel_loop(0, BPB * S_BLK, step=1, unroll=8)
            def seq_row(i):
                vvec = plsc.load_gather(
                    idx1, [jnp.full((16,), p * BPB * S_BLK + i, jnp.int32)])
                base = vvec * EMB + col
                ci = lax.rem(i, S_BLK)
                for g in range(EMB // 16):
                    sl = pl.ds(g * 16, 16)
                    row = plsc.load_gather(table1, [base + g * 16])
                    out_v[k, i, sl] = row + comb_v[ci, sl]
            start_out(p, k)
        return carry
    lax.fori_loop(0, NPAIR // NBUF, batch_pair, 0)

    for k in range(NBUF):
        wait_out(NPAIR - NBUF + k, k)


@jax.jit
def kernel(vertices, vtx_table, pos_table, dim_table):
    vert1 = vertices.astype(jnp.int32).reshape(-1)
    vtx1 = (vtx_table * SCALE).reshape(-1)
    mesh = plsc.VectorSubcoreMesh(core_axis_name="c", subcore_axis_name="s")
    f = functools.partial(
        pl.kernel,
        mesh=mesh,
        out_type=jax.ShapeDtypeStruct((BATCH, SEQ, EMB), jnp.float32),
        scratch_types=[
            pltpu.VMEM((B_BLK * S_BLK,), jnp.int32),
            pltpu.VMEM((NUM_VTX * EMB,), jnp.float32),
            pltpu.VMEM((NUM_DIM, EMB), jnp.float32),
            pltpu.VMEM((S_BLK, EMB), jnp.float32),
            pltpu.VMEM((NBUF, BPB * S_BLK, EMB), jnp.float32),
        ] + [pltpu.SemaphoreType.DMA] * 3,
        compiler_params=pltpu.CompilerParams(needs_layout_passes=False),
    )(_body)
    return f(vert1, vtx1, pos_table, dim_table)


# speedup vs baseline: 1.5136x; 1.4621x over previous
import functools

import jax
import jax.numpy as jnp
from jax import lax
from jax.experimental import pallas as pl
from jax.experimental.pallas import tpu as pltpu
from jax.experimental.pallas import tpu_sc as plsc

NUM_VTX = 227
NUM_DIM = 3
EMB = 128
BATCH = 256
SEQ = 2048
SCALE = float(EMB) ** 0.5

NC = 2
NS = 16
NW = NC * NS
S_BLK = 128
B_BLK = BATCH // 2
NBUF = 2
BPB = 2
NPAIR = B_BLK // BPB


def _body(vert_hbm, vtx_hbm, pos_hbm, dim_hbm, out_hbm,
          idx1, table1, dim_v, comb_v, comb_p, out_v, isem, o0, o1):
    osem = [o0, o1]
    wid = lax.axis_index("s") * NC + lax.axis_index("c")
    j = lax.rem(wid, 16)
    h = wid // 16
    s0 = j * S_BLK
    b0 = h * B_BLK

    pltpu.sync_copy(vtx_hbm, table1)
    pltpu.sync_copy(dim_hbm, dim_v)
    pltpu.sync_copy(pos_hbm.at[pl.ds(s0, S_BLK)], comb_v)

    def idx_row(b, carry):
        pltpu.async_copy(
            vert_hbm.at[pl.ds((b0 + b) * SEQ + s0, S_BLK)],
            idx1.at[pl.ds(b * S_BLK, S_BLK)], isem)
        return carry
    lax.fori_loop(0, B_BLK, idx_row, 0)

    def idx_drain(b, carry):
        pltpu.make_async_copy(
            vert_hbm.at[pl.ds(b0 * SEQ + s0, S_BLK)],
            idx1.at[pl.ds(0, S_BLK)], isem).wait()
        return carry
    lax.fori_loop(0, B_BLK, idx_drain, 0)

    def comb_row(i, carry):
        r = lax.rem(s0 + i, NUM_DIM)
        w0 = jnp.full((16,), jnp.where(r == 0, SCALE, 0.0), jnp.float32)
        w1 = jnp.full((16,), jnp.where(r == 1, SCALE, 0.0), jnp.float32)
        w2 = jnp.full((16,), jnp.where(r == 2, SCALE, 0.0), jnp.float32)
        for g in range(EMB // 16):
            sl = pl.ds(g * 16, 16)
            comb_v[i, sl] = (comb_v[i, sl] * SCALE
                             + dim_v[0, sl] * w0
                             + dim_v[1, sl] * w1
                             + dim_v[2, sl] * w2)
        return carry
    lax.fori_loop(0, S_BLK, comb_row, 0)

    def comb_pack(i, carry):
        for g2 in range(EMB // 32):
            a = comb_v[i, pl.ds(g2 * 32, 16)]
            b = comb_v[i, pl.ds(g2 * 32 + 16, 16)]
            w = plsc.pack(a, b, format=plsc.PackFormat.INTERLEAVED)
            comb_p[i, pl.ds(g2 * 16, 16)] = plsc.bitcast(w, jnp.int32)
        return carry
    lax.fori_loop(0, S_BLK, comb_pack, 0)

    col = lax.iota(jnp.int32, 16)

    def start_out(p, k):
        for r in range(BPB):
            pltpu.async_copy(
                out_v.at[k, pl.ds(r * S_BLK, S_BLK)],
                out_hbm.at[b0 + p * BPB + r, pl.ds(s0, S_BLK)], osem[k])

    def wait_out(p, k):
        for r in range(BPB):
            pltpu.make_async_copy(
                out_v.at[k, pl.ds(r * S_BLK, S_BLK)],
                out_hbm.at[b0 + p * BPB + r, pl.ds(s0, S_BLK)],
                osem[k]).wait()

    def batch_pair(bp, carry):
        for k in range(NBUF):
            p = bp * NBUF + k

            @pl.when(p >= NBUF)
            def _():
                wait_out(p - NBUF, k)

            @plsc.parallel_loop(0, BPB * S_BLK, step=1, unroll=8)
            def seq_row(i):
                vvec = plsc.load_gather(
                    idx1, [jnp.full((16,), p * BPB * S_BLK + i, jnp.int32)])
                base = vvec * (EMB // 2) + col
                ci = lax.rem(i, S_BLK)
                for g2 in range(EMB // 32):
                    tw = plsc.load_gather(table1, [base + g2 * 16])
                    cw = comb_p[ci, pl.ds(g2 * 16, 16)]
                    ta, tb = plsc.unpack(
                        plsc.bitcast(tw, jnp.bfloat16),
                        format=plsc.PackFormat.INTERLEAVED)
                    ca, cb = plsc.unpack(
                        plsc.bitcast(cw, jnp.bfloat16),
                        format=plsc.PackFormat.INTERLEAVED)
                    out_v[k, i, pl.ds(g2 * 32, 16)] = ta + ca
                    out_v[k, i, pl.ds(g2 * 32 + 16, 16)] = tb + cb
            start_out(p, k)
        return carry
    lax.fori_loop(0, NPAIR // NBUF, batch_pair, 0)

    for k in range(NBUF):
        wait_out(NPAIR - NBUF + k, k)


@jax.jit
def kernel(vertices, vtx_table, pos_table, dim_table):
    vert1 = vertices.astype(jnp.int32).reshape(-1)
    t = (vtx_table * SCALE).astype(jnp.bfloat16)
    t = t.reshape(NUM_VTX, EMB // 32, 2, 16).transpose(0, 1, 3, 2)
    vtx1 = jax.lax.bitcast_convert_type(t, jnp.int32).reshape(-1)
    mesh = plsc.VectorSubcoreMesh(core_axis_name="c", subcore_axis_name="s")
    f = functools.partial(
        pl.kernel,
        mesh=mesh,
        out_type=jax.ShapeDtypeStruct((BATCH, SEQ, EMB), jnp.float32),
        scratch_types=[
            pltpu.VMEM((B_BLK * S_BLK,), jnp.int32),
            pltpu.VMEM((NUM_VTX * EMB // 2,), jnp.int32),
            pltpu.VMEM((NUM_DIM, EMB), jnp.float32),
            pltpu.VMEM((S_BLK, EMB), jnp.float32),
            pltpu.VMEM((S_BLK, EMB // 2), jnp.int32),
            pltpu.VMEM((NBUF, BPB * S_BLK, EMB), jnp.float32),
        ] + [pltpu.SemaphoreType.DMA] * 3,
        compiler_params=pltpu.CompilerParams(needs_layout_passes=False),
    )(_body)
    return f(vert1, vtx1, pos_table, dim_table)
